# baseline (speedup 1.0000x reference)
import functools

import jax
import jax.numpy as jnp
from jax.experimental import pallas as pl
from jax.experimental.pallas import tpu as pltpu

_NEG = -1e30


def _fused_kernel(y_ref, a_ref, w1_ref, b1_ref, w2_ref, b2_ref, w3_ref,
                  mt_ref, mc_ref, yhat_ref,
                  m_s, s_s, v_s, c_s, af_s, mcs_s, *, nb, pcls):
    j = pl.program_id(0)

    @pl.when(j == 0)
    def _init():
        m_s[...] = jnp.full(m_s.shape, _NEG, jnp.float32)
        s_s[...] = jnp.zeros(s_s.shape, jnp.float32)
        v_s[...] = jnp.zeros(v_s.shape, jnp.float32)
        c_s[...] = jnp.zeros(c_s.shape, jnp.float32)

    cdim, bh, wdim = y_ref.shape
    npix = bh * wdim

    @pl.when(j < nb)
    def _stats():
        y = y_ref[...].astype(jnp.bfloat16).reshape(cdim, npix)
        a = a_ref[...].reshape(pcls, npix)
        af_s[:, pl.ds(j * npix, npix)] = a

        h = jnp.maximum(
            jnp.dot(w1_ref[...].astype(jnp.bfloat16), y,
                    preferred_element_type=jnp.float32)
            + b1_ref[...], 0.0).astype(jnp.bfloat16)
        h = jnp.maximum(
            jnp.dot(w2_ref[...].astype(jnp.bfloat16), h,
                    preferred_element_type=jnp.float32)
            + b2_ref[...], 0.0).astype(jnp.bfloat16)
        logits = jnp.dot(w3_ref[...].astype(jnp.bfloat16), h,
                         preferred_element_type=jnp.float32)

        amax = jnp.max(a, axis=0, keepdims=True)
        iota = jax.lax.broadcasted_iota(jnp.int32, a.shape, 0)
        idx = jnp.min(jnp.where(a == amax, iota, pcls), axis=0, keepdims=True)
        onehot = iota == idx

        lmask = jnp.where(onehot, logits, _NEG)
        bm = jnp.max(lmask, axis=1, keepdims=True)
        m_old = m_s[:, 0:1]
        m_new = jnp.maximum(m_old, bm)
        alpha = jnp.exp(m_old - m_new)
        p = jnp.exp(lmask - m_new)
        s_new = s_s[:, 0:1] * alpha + jnp.sum(p, axis=1, keepdims=True)
        v_new = v_s[...] * alpha + jax.lax.dot_general(
            p.astype(jnp.bfloat16), y, (((1,), (1,)), ((), ())),
            preferred_element_type=jnp.float32)
        c_new = c_s[:, 0:1] + jnp.sum(onehot.astype(jnp.float32), axis=1,
                                      keepdims=True)

        m_s[...] = jnp.broadcast_to(m_new, m_s.shape)
        s_s[...] = jnp.broadcast_to(s_new, s_s.shape)
        v_s[...] = v_new
        c_s[...] = jnp.broadcast_to(c_new, c_s.shape)

        @pl.when(j == nb - 1)
        def _fin():
            w = v_s[...] / s_s[:, 0:1]
            mb = jnp.maximum(mt_ref[...], 0.0)
            col = jnp.where(c_s[:, 0:1] > 10.0, 0.5 * w + 0.5 * mb, mb)
            mc = jnp.clip(col, 0.0, 2.0)
            mcs_s[...] = mc
            mc_ref[...] = mc

    @pl.when(j >= nb)
    def _recon():
        k = j - nb
        a = af_s[:, pl.ds(k * npix, npix)]
        res = jax.lax.dot_general(
            mcs_s[...], a, (((0,), (0,)), ((), ())),
            preferred_element_type=jnp.float32)
        yhat_ref[...] = res.reshape(cdim, bh, wdim)


@jax.jit
def kernel(abundances, Y, M, W1, b1, W2, b2, W3, b3):
    B, P, H, W_ = abundances.shape
    C = Y.shape[1]
    A3 = abundances.reshape(P, H, W_)
    Y3 = Y.reshape(C, H, W_)
    D1 = W1.shape[0]

    bh = 16
    nb = H // bh
    npix = bh * W_

    mc_t, yhat3 = pl.pallas_call(
        functools.partial(_fused_kernel, nb=nb, pcls=P),
        grid=(2 * nb,),
        in_specs=[
            pl.BlockSpec((C, bh, W_), lambda j: (0, jnp.minimum(j, nb - 1), 0)),
            pl.BlockSpec((P, bh, W_), lambda j: (0, jnp.minimum(j, nb - 1), 0)),
            pl.BlockSpec((D1, C), lambda j: (0, 0)),
            pl.BlockSpec((D1, 1), lambda j: (0, 0)),
            pl.BlockSpec((D1, D1), lambda j: (0, 0)),
            pl.BlockSpec((D1, 1), lambda j: (0, 0)),
            pl.BlockSpec((1, D1), lambda j: (0, 0)),
            pl.BlockSpec((P, C), lambda j: (0, 0)),
        ],
        out_specs=[
            pl.BlockSpec((P, C), lambda j: (0, 0)),
            pl.BlockSpec((C, bh, W_), lambda j: (0, jnp.maximum(j - nb, 0), 0)),
        ],
        out_shape=[
            jax.ShapeDtypeStruct((P, C), jnp.float32),
            jax.ShapeDtypeStruct((C, H, W_), jnp.float32),
        ],
        scratch_shapes=[
            pltpu.VMEM((P, 128), jnp.float32),
            pltpu.VMEM((P, 128), jnp.float32),
            pltpu.VMEM((P, C), jnp.float32),
            pltpu.VMEM((P, 128), jnp.float32),
            pltpu.VMEM((P, H * W_), jnp.float32),
            pltpu.VMEM((P, C), jnp.float32),
        ],
    )(Y3, A3, W1, b1.reshape(D1, 1), W2, b2.reshape(D1, 1), W3, M.T)

    return yhat3.reshape(B, C, H, W_), mc_t.T


# speedup vs baseline: 1.1540x; 1.1540x over previous
import functools

import jax
import jax.numpy as jnp
from jax.experimental import pallas as pl
from jax.experimental.pallas import tpu as pltpu

_NEG = -1e30


def _fused_kernel(y_ref, a_ref, w1_ref, b1_ref, w2_ref, b2_ref, w3_ref,
                  mt_ref, mc_ref, yhat_ref,
                  m_s, s_s, v_s, c_s, af_s, mcs_s, yb_s, dma_sem,
                  *, nb, pcls, bh):
    j = pl.program_id(0)

    @pl.when(j == 0)
    def _init():
        m_s[...] = jnp.full(m_s.shape, _NEG, jnp.float32)
        s_s[...] = jnp.zeros(s_s.shape, jnp.float32)
        v_s[...] = jnp.zeros(v_s.shape, jnp.float32)
        c_s[...] = jnp.zeros(c_s.shape, jnp.float32)

    cdim = y_ref.shape[0]
    wdim = y_ref.shape[2]
    npix = bh * wdim

    def _fetch(blk, slot):
        for r in range(bh):
            pltpu.make_async_copy(
                y_ref.at[:, blk * bh + r, :],
                yb_s.at[:, pl.ds(slot * npix + r * wdim, wdim)],
                dma_sem.at[slot],
            ).start()

    def _wait(blk, slot):
        for r in range(bh):
            pltpu.make_async_copy(
                y_ref.at[:, blk * bh + r, :],
                yb_s.at[:, pl.ds(slot * npix + r * wdim, wdim)],
                dma_sem.at[slot],
            ).wait()

    @pl.when(j == 0)
    def _prologue():
        _fetch(0, 0)

    @pl.when(j < nb)
    def _stats():
        @pl.when(j + 1 < nb)
        def _prefetch():
            _fetch(j + 1, (j + 1) % 2)

        _wait(j, j % 2)
        slot = j % 2
        y = yb_s[:, pl.ds(slot * npix, npix)].astype(jnp.bfloat16)
        a = a_ref[...].reshape(pcls, npix)
        af_s[:, pl.ds(j * npix, npix)] = a

        h = jnp.maximum(
            jnp.dot(w1_ref[...].astype(jnp.bfloat16), y,
                    preferred_element_type=jnp.float32)
            + b1_ref[...], 0.0).astype(jnp.bfloat16)
        h = jnp.maximum(
            jnp.dot(w2_ref[...].astype(jnp.bfloat16), h,
                    preferred_element_type=jnp.float32)
            + b2_ref[...], 0.0).astype(jnp.bfloat16)
        logits = jnp.dot(w3_ref[...].astype(jnp.bfloat16), h,
                         preferred_element_type=jnp.float32)

        amax = jnp.max(a, axis=0, keepdims=True)
        iota = jax.lax.broadcasted_iota(jnp.int32, a.shape, 0)
        idx = jnp.min(jnp.where(a == amax, iota, pcls), axis=0, keepdims=True)
        onehot = iota == idx

        lmask = jnp.where(onehot, logits, _NEG)
        bm = jnp.max(lmask, axis=1, keepdims=True)
        m_old = m_s[:, 0:1]
        m_new = jnp.maximum(m_old, bm)
        alpha = jnp.exp(m_old - m_new)
        p = jnp.exp(lmask - m_new)
        s_new = s_s[:, 0:1] * alpha + jnp.sum(p, axis=1, keepdims=True)
        v_new = v_s[...] * alpha + jax.lax.dot_general(
            p.astype(jnp.bfloat16), y, (((1,), (1,)), ((), ())),
            preferred_element_type=jnp.float32)
        c_new = c_s[:, 0:1] + jnp.sum(onehot.astype(jnp.float32), axis=1,
                                      keepdims=True)

        m_s[...] = jnp.broadcast_to(m_new, m_s.shape)
        s_s[...] = jnp.broadcast_to(s_new, s_s.shape)
        v_s[...] = v_new
        c_s[...] = jnp.broadcast_to(c_new, c_s.shape)

        @pl.when(j == nb - 1)
        def _fin():
            w = v_s[...] / s_s[:, 0:1]
            mb = jnp.maximum(mt_ref[...], 0.0)
            col = jnp.where(c_s[:, 0:1] > 10.0, 0.5 * w + 0.5 * mb, mb)
            mc = jnp.clip(col, 0.0, 2.0)
            mcs_s[...] = mc
            mc_ref[...] = mc

    @pl.when(j >= nb)
    def _recon():
        k = j - nb
        a = af_s[:, pl.ds(k * npix, npix)]
        res = jax.lax.dot_general(
            mcs_s[...], a, (((0,), (0,)), ((), ())),
            preferred_element_type=jnp.float32)
        yhat_ref[...] = res.reshape(cdim, bh, wdim)


@jax.jit
def kernel(abundances, Y, M, W1, b1, W2, b2, W3, b3):
    B, P, H, W_ = abundances.shape
    C = Y.shape[1]
    A3 = abundances.reshape(P, H, W_)
    Y3 = Y.reshape(C, H, W_)
    D1 = W1.shape[0]

    bh = 16
    nb = H // bh
    npix = bh * W_

    mc_t, yhat3 = pl.pallas_call(
        functools.partial(_fused_kernel, nb=nb, pcls=P, bh=bh),
        grid=(2 * nb,),
        in_specs=[
            pl.BlockSpec(memory_space=pltpu.MemorySpace.HBM),
            pl.BlockSpec((P, bh, W_), lambda j: (0, jnp.minimum(j, nb - 1), 0)),
            pl.BlockSpec((D1, C), lambda j: (0, 0)),
            pl.BlockSpec((D1, 1), lambda j: (0, 0)),
            pl.BlockSpec((D1, D1), lambda j: (0, 0)),
            pl.BlockSpec((D1, 1), lambda j: (0, 0)),
            pl.BlockSpec((1, D1), lambda j: (0, 0)),
            pl.BlockSpec((P, C), lambda j: (0, 0)),
        ],
        out_specs=[
            pl.BlockSpec((P, C), lambda j: (0, 0)),
            pl.BlockSpec((C, bh, W_), lambda j: (0, jnp.maximum(j - nb, 0), 0)),
        ],
        out_shape=[
            jax.ShapeDtypeStruct((P, C), jnp.float32),
            jax.ShapeDtypeStruct((C, H, W_), jnp.float32),
        ],
        scratch_shapes=[
            pltpu.VMEM((P, 128), jnp.float32),
            pltpu.VMEM((P, 128), jnp.float32),
            pltpu.VMEM((P, C), jnp.float32),
            pltpu.VMEM((P, 128), jnp.float32),
            pltpu.VMEM((P, H * W_), jnp.float32),
            pltpu.VMEM((P, C), jnp.float32),
            pltpu.VMEM((C, 2 * npix), jnp.float32),
            pltpu.SemaphoreType.DMA((2,)),
        ],
    )(Y3, A3, W1, b1.reshape(D1, 1), W2, b2.reshape(D1, 1), W3, M.T)

    return yhat3.reshape(B, C, H, W_), mc_t.T


# speedup vs baseline: 1.2546x; 1.0872x over previous
import functools

import jax
import jax.numpy as jnp
from jax.experimental import pallas as pl
from jax.experimental.pallas import tpu as pltpu

_NEG = -1e30


def _fused_kernel(y_ref, a_ref, w1_ref, b1_ref, w2_ref, b2_ref, w3_ref,
                  mt_ref, mc_ref, yhat_ref,
                  m_s, s_s, v_s, c_s, af_s, mcs_s, yb_s, dma_sem,
                  rb_s, out_sem, *, nb, pcls, bh):
    j = pl.program_id(0)

    @pl.when(j == 0)
    def _init():
        m_s[...] = jnp.full(m_s.shape, _NEG, jnp.float32)
        s_s[...] = jnp.zeros(s_s.shape, jnp.float32)
        v_s[...] = jnp.zeros(v_s.shape, jnp.float32)
        c_s[...] = jnp.zeros(c_s.shape, jnp.float32)

    cdim = y_ref.shape[0]
    wdim = y_ref.shape[2]
    npix = bh * wdim

    def _fetch(blk, slot):
        for r in range(bh):
            pltpu.make_async_copy(
                y_ref.at[:, blk * bh + r, :],
                yb_s.at[:, pl.ds(slot * npix + r * wdim, wdim)],
                dma_sem.at[slot],
            ).start()

    def _wait(blk, slot):
        for r in range(bh):
            pltpu.make_async_copy(
                y_ref.at[:, blk * bh + r, :],
                yb_s.at[:, pl.ds(slot * npix + r * wdim, wdim)],
                dma_sem.at[slot],
            ).wait()

    @pl.when(j == 0)
    def _prologue():
        _fetch(0, 0)

    @pl.when(j < nb)
    def _stats():
        @pl.when(j + 1 < nb)
        def _prefetch():
            _fetch(j + 1, (j + 1) % 2)

        _wait(j, j % 2)
        slot = j % 2
        y = yb_s[:, pl.ds(slot * npix, npix)].astype(jnp.bfloat16)
        a = a_ref[...].reshape(pcls, npix)
        af_s[:, pl.ds(j * npix, npix)] = a

        h = jnp.maximum(
            jnp.dot(w1_ref[...].astype(jnp.bfloat16), y,
                    preferred_element_type=jnp.float32)
            + b1_ref[...], 0.0).astype(jnp.bfloat16)
        h = jnp.maximum(
            jnp.dot(w2_ref[...].astype(jnp.bfloat16), h,
                    preferred_element_type=jnp.float32)
            + b2_ref[...], 0.0).astype(jnp.bfloat16)
        logits = jnp.dot(w3_ref[...].astype(jnp.bfloat16), h,
                         preferred_element_type=jnp.float32)

        amax = jnp.max(a, axis=0, keepdims=True)
        iota = jax.lax.broadcasted_iota(jnp.int32, a.shape, 0)
        idx = jnp.min(jnp.where(a == amax, iota, pcls), axis=0, keepdims=True)
        onehot = iota == idx

        lmask = jnp.where(onehot, logits, _NEG)
        bm = jnp.max(lmask, axis=1, keepdims=True)
        m_old = m_s[:, 0:1]
        m_new = jnp.maximum(m_old, bm)
        alpha = jnp.exp(m_old - m_new)
        p = jnp.exp(lmask - m_new)
        s_new = s_s[:, 0:1] * alpha + jnp.sum(p, axis=1, keepdims=True)
        v_new = v_s[...] * alpha + jax.lax.dot_general(
            p.astype(jnp.bfloat16), y, (((1,), (1,)), ((), ())),
            preferred_element_type=jnp.float32)
        c_new = c_s[:, 0:1] + jnp.sum(onehot.astype(jnp.float32), axis=1,
                                      keepdims=True)

        m_s[...] = jnp.broadcast_to(m_new, m_s.shape)
        s_s[...] = jnp.broadcast_to(s_new, s_s.shape)
        v_s[...] = v_new
        c_s[...] = jnp.broadcast_to(c_new, c_s.shape)

        @pl.when(j == nb - 1)
        def _fin():
            w = v_s[...] / s_s[:, 0:1]
            mb = jnp.maximum(mt_ref[...], 0.0)
            col = jnp.where(c_s[:, 0:1] > 10.0, 0.5 * w + 0.5 * mb, mb)
            mc = jnp.clip(col, 0.0, 2.0)
            mcs_s[...] = mc
            mc_ref[...] = mc

    def _ocopies(blk, slot):
        return [pltpu.make_async_copy(
            rb_s.at[:, pl.ds(slot * npix + r * wdim, wdim)],
            yhat_ref.at[:, blk * bh + r, :],
            out_sem.at[slot]) for r in range(bh)]

    @pl.when(j >= nb)
    def _recon():
        k = j - nb
        slot = k % 2

        @pl.when(k >= 2)
        def _reclaim():
            for c in _ocopies(k - 2, slot):
                c.wait()

        a = af_s[:, pl.ds(k * npix, npix)]
        res = jax.lax.dot_general(
            mcs_s[...], a, (((0,), (0,)), ((), ())),
            preferred_element_type=jnp.float32)
        rb_s[:, pl.ds(slot * npix, npix)] = res
        for c in _ocopies(k, slot):
            c.start()

        @pl.when(k == nb - 1)
        def _drain():
            for c in _ocopies(k - 1, 1 - slot):
                c.wait()
            for c in _ocopies(k, slot):
                c.wait()


@jax.jit
def kernel(abundances, Y, M, W1, b1, W2, b2, W3, b3):
    B, P, H, W_ = abundances.shape
    C = Y.shape[1]
    A3 = abundances.reshape(P, H, W_)
    Y3 = Y.reshape(C, H, W_)
    D1 = W1.shape[0]

    bh = 16
    nb = H // bh
    npix = bh * W_

    mc_t, yhat3 = pl.pallas_call(
        functools.partial(_fused_kernel, nb=nb, pcls=P, bh=bh),
        grid=(2 * nb,),
        in_specs=[
            pl.BlockSpec(memory_space=pltpu.MemorySpace.HBM),
            pl.BlockSpec((P, bh, W_), lambda j: (0, jnp.minimum(j, nb - 1), 0)),
            pl.BlockSpec((D1, C), lambda j: (0, 0)),
            pl.BlockSpec((D1, 1), lambda j: (0, 0)),
            pl.BlockSpec((D1, D1), lambda j: (0, 0)),
            pl.BlockSpec((D1, 1), lambda j: (0, 0)),
            pl.BlockSpec((1, D1), lambda j: (0, 0)),
            pl.BlockSpec((P, C), lambda j: (0, 0)),
        ],
        out_specs=[
            pl.BlockSpec((P, C), lambda j: (0, 0)),
            pl.BlockSpec(memory_space=pltpu.MemorySpace.HBM),
        ],
        out_shape=[
            jax.ShapeDtypeStruct((P, C), jnp.float32),
            jax.ShapeDtypeStruct((C, H, W_), jnp.float32),
        ],
        scratch_shapes=[
            pltpu.VMEM((P, 128), jnp.float32),
            pltpu.VMEM((P, 128), jnp.float32),
            pltpu.VMEM((P, C), jnp.float32),
            pltpu.VMEM((P, 128), jnp.float32),
            pltpu.VMEM((P, H * W_), jnp.float32),
            pltpu.VMEM((P, C), jnp.float32),
            pltpu.VMEM((C, 2 * npix), jnp.float32),
            pltpu.SemaphoreType.DMA((2,)),
            pltpu.VMEM((C, 2 * npix), jnp.float32),
            pltpu.SemaphoreType.DMA((2,)),
        ],
    )(Y3, A3, W1, b1.reshape(D1, 1), W2, b2.reshape(D1, 1), W3, M.T)

    return yhat3.reshape(B, C, H, W_), mc_t.T


# speedup vs baseline: 1.2656x; 1.0087x over previous
import functools

import jax
import jax.numpy as jnp
from jax.experimental import pallas as pl
from jax.experimental.pallas import tpu as pltpu

_NEG = -1e30


def _fused_kernel(y_ref, a_ref, w1_ref, b1_ref, w2_ref, b2_ref, w3_ref,
                  mt_ref, mc_ref, yhat_ref,
                  m_s, s_s, v_s, c_s, af_s, mcs_s, yb_s, dma_sem,
                  rb_s, out_sem, *, nb, pcls, bh):
    j = pl.program_id(0)

    @pl.when(j == 0)
    def _init():
        m_s[...] = jnp.full(m_s.shape, _NEG, jnp.float32)
        s_s[...] = jnp.zeros(s_s.shape, jnp.float32)
        v_s[...] = jnp.zeros(v_s.shape, jnp.float32)
        c_s[...] = jnp.zeros(c_s.shape, jnp.float32)

    cdim = y_ref.shape[0]
    wdim = y_ref.shape[2]
    npix = bh * wdim

    def _in_copies(blk, slot):
        cps = []
        for r in range(bh):
            cps.append(pltpu.make_async_copy(
                y_ref.at[:, blk * bh + r, :],
                yb_s.at[:, pl.ds(slot * npix + r * wdim, wdim)],
                dma_sem.at[slot]))
            cps.append(pltpu.make_async_copy(
                a_ref.at[:, blk * bh + r, :],
                af_s.at[:, pl.ds((blk * bh + r) * wdim, wdim)],
                dma_sem.at[slot]))
        return cps

    def _fetch(blk, slot):
        for c in _in_copies(blk, slot):
            c.start()

    def _wait(blk, slot):
        for c in _in_copies(blk, slot):
            c.wait()

    @pl.when(j == 0)
    def _prologue():
        _fetch(0, 0)

    @pl.when(j < nb)
    def _stats():
        @pl.when(j + 1 < nb)
        def _prefetch():
            _fetch(j + 1, (j + 1) % 2)

        _wait(j, j % 2)
        slot = j % 2
        y = yb_s[:, pl.ds(slot * npix, npix)].astype(jnp.bfloat16)
        a = af_s[:, pl.ds(j * npix, npix)]

        h = jnp.maximum(
            jnp.dot(w1_ref[...].astype(jnp.bfloat16), y,
                    preferred_element_type=jnp.float32)
            + b1_ref[...], 0.0).astype(jnp.bfloat16)
        h = jnp.maximum(
            jnp.dot(w2_ref[...].astype(jnp.bfloat16), h,
                    preferred_element_type=jnp.float32)
            + b2_ref[...], 0.0).astype(jnp.bfloat16)
        logits = jnp.dot(w3_ref[...].astype(jnp.bfloat16), h,
                         preferred_element_type=jnp.float32)

        amax = jnp.max(a, axis=0, keepdims=True)
        iota = jax.lax.broadcasted_iota(jnp.int32, a.shape, 0)
        idx = jnp.min(jnp.where(a == amax, iota, pcls), axis=0, keepdims=True)
        onehot = iota == idx

        lmask = jnp.where(onehot, logits, _NEG)
        bm = jnp.max(lmask, axis=1, keepdims=True)
        m_old = m_s[:, 0:1]
        m_new = jnp.maximum(m_old, bm)
        alpha = jnp.exp(m_old - m_new)
        p = jnp.exp(lmask - m_new)
        s_new = s_s[:, 0:1] * alpha + jnp.sum(p, axis=1, keepdims=True)
        v_new = v_s[...] * alpha + jax.lax.dot_general(
            p.astype(jnp.bfloat16), y, (((1,), (1,)), ((), ())),
            preferred_element_type=jnp.float32)
        c_new = c_s[:, 0:1] + jnp.sum(onehot.astype(jnp.float32), axis=1,
                                      keepdims=True)

        m_s[...] = jnp.broadcast_to(m_new, m_s.shape)
        s_s[...] = jnp.broadcast_to(s_new, s_s.shape)
        v_s[...] = v_new
        c_s[...] = jnp.broadcast_to(c_new, c_s.shape)

        @pl.when(j == nb - 1)
        def _fin():
            w = v_s[...] / s_s[:, 0:1]
            mb = jnp.maximum(mt_ref[...], 0.0)
            col = jnp.where(c_s[:, 0:1] > 10.0, 0.5 * w + 0.5 * mb, mb)
            mc = jnp.clip(col, 0.0, 2.0)
            mcs_s[...] = mc
            mc_ref[...] = mc

    def _ocopies(blk, slot):
        return [pltpu.make_async_copy(
            rb_s.at[:, pl.ds(slot * npix + r * wdim, wdim)],
            yhat_ref.at[:, blk * bh + r, :],
            out_sem.at[slot]) for r in range(bh)]

    @pl.when(j >= nb)
    def _recon():
        k = j - nb
        slot = k % 2

        @pl.when(k >= 2)
        def _reclaim():
            for c in _ocopies(k - 2, slot):
                c.wait()

        a = af_s[:, pl.ds(k * npix, npix)]
        res = jax.lax.dot_general(
            mcs_s[...], a, (((0,), (0,)), ((), ())),
            preferred_element_type=jnp.float32)
        rb_s[:, pl.ds(slot * npix, npix)] = res
        for c in _ocopies(k, slot):
            c.start()

        @pl.when(k == nb - 1)
        def _drain():
            for c in _ocopies(k - 1, 1 - slot):
                c.wait()
            for c in _ocopies(k, slot):
                c.wait()


@jax.jit
def kernel(abundances, Y, M, W1, b1, W2, b2, W3, b3):
    B, P, H, W_ = abundances.shape
    C = Y.shape[1]
    A3 = abundances.reshape(P, H, W_)
    Y3 = Y.reshape(C, H, W_)
    D1 = W1.shape[0]

    bh = 16
    nb = H // bh
    npix = bh * W_

    mc_t, yhat3 = pl.pallas_call(
        functools.partial(_fused_kernel, nb=nb, pcls=P, bh=bh),
        grid=(2 * nb,),
        in_specs=[
            pl.BlockSpec(memory_space=pltpu.MemorySpace.HBM),
            pl.BlockSpec(memory_space=pltpu.MemorySpace.HBM),
            pl.BlockSpec((D1, C), lambda j: (0, 0)),
            pl.BlockSpec((D1, 1), lambda j: (0, 0)),
            pl.BlockSpec((D1, D1), lambda j: (0, 0)),
            pl.BlockSpec((D1, 1), lambda j: (0, 0)),
            pl.BlockSpec((1, D1), lambda j: (0, 0)),
            pl.BlockSpec((P, C), lambda j: (0, 0)),
        ],
        out_specs=[
            pl.BlockSpec((P, C), lambda j: (0, 0)),
            pl.BlockSpec(memory_space=pltpu.MemorySpace.HBM),
        ],
        out_shape=[
            jax.ShapeDtypeStruct((P, C), jnp.float32),
            jax.ShapeDtypeStruct((C, H, W_), jnp.float32),
        ],
        scratch_shapes=[
            pltpu.VMEM((P, 128), jnp.float32),
            pltpu.VMEM((P, 128), jnp.float32),
            pltpu.VMEM((P, C), jnp.float32),
            pltpu.VMEM((P, 128), jnp.float32),
            pltpu.VMEM((P, H * W_), jnp.float32),
            pltpu.VMEM((P, C), jnp.float32),
            pltpu.VMEM((C, 2 * npix), jnp.float32),
            pltpu.SemaphoreType.DMA((2,)),
            pltpu.VMEM((C, 2 * npix), jnp.float32),
            pltpu.SemaphoreType.DMA((2,)),
        ],
    )(Y3, A3, W1, b1.reshape(D1, 1), W2, b2.reshape(D1, 1), W3, M.T)

    return yhat3.reshape(B, C, H, W_), mc_t.T
